# Initial kernel scaffold; baseline (speedup 1.0000x reference)
import jax
import jax.numpy as jnp
from jax.experimental import pallas as pl


def kernel(edge_index, edge_type, use_cuda, dynamic_emb, emb_rel, w1, weight_neighbor, loop_weight, w_ih, w_hh, b_ih, b_hh):
    raise NotImplementedError("write your pallas kernel here")


# speedup vs baseline: 3.6899x; 3.6899x over previous
import functools

import jax
import jax.numpy as jnp
from jax import lax
from jax.experimental import pallas as pl
from jax.experimental.pallas import tpu as pltpu
from jax.experimental.pallas import tpu_sc as plsc

NE = 10000
H = 128
EDGES = 320000
R2 = 400
TSTEPS = 3

NC = 2
NS = 16
NW = NC * NS
EW = EDGES // NW
CH = 80
NCHUNK = EW // CH
ZB = 104

_REL_PER_SUB = 40
_ENT_PER_SUB = 624

NEG_SLOPE = (1.0 / 8.0 + 1.0 / 3.0) / 2.0


def _sc_mesh():
    return plsc.VectorSubcoreMesh(
        core_axis_name="c", subcore_axis_name="s", num_cores=NC, num_subcores=NS
    )


def _zero_vmem(ref, nrows, ncols):
    def row(i, _):
        def col(j, __):
            ref[i, pl.ds(j * 16, 16)] = jnp.zeros((16,), jnp.float32)
            return 0
        lax.fori_loop(0, ncols // 16, col, 0)
        return 0
    lax.fori_loop(0, nrows, row, 0)


def _ent_copy(s, fn):
    for r in range(_ENT_PER_SUB // ZB):
        fn(s * _ENT_PER_SUB + r * ZB, ZB)

    @pl.when(s == NS - 1)
    def _tail():
        fn(NS * _ENT_PER_SUB, NE - NS * _ENT_PER_SUB)


def _fused_sweep_body(table, src, et, dst, seg_out, cnt_out, s_out,
                      gidx_v, eidx_v, didx_v, rows_v, ones_v, zbuf_v,
                      seg_sh, cnt_sh, s_sh, sem):
    c = lax.axis_index("c")
    s = lax.axis_index("s")
    wid = c * NS + s
    base = wid * EW

    _zero_vmem(zbuf_v, ZB, H)

    def fill_ones(i, _):
        def col(j, __):
            ones_v[i, pl.ds(j * 16, 16)] = jnp.ones((16,), jnp.float32)
            return 0
        lax.fori_loop(0, H // 16, col, 0)
        return 0
    lax.fori_loop(0, CH, fill_ones, 0)

    @pl.when(s < R2 // _REL_PER_SUB)
    def _zrel():
        pltpu.sync_copy(zbuf_v.at[pl.ds(0, _REL_PER_SUB)],
                        seg_sh.at[pl.ds(s * _REL_PER_SUB, _REL_PER_SUB)])
        pltpu.sync_copy(zbuf_v.at[pl.ds(0, _REL_PER_SUB)],
                        cnt_sh.at[pl.ds(s * _REL_PER_SUB, _REL_PER_SUB)])

    def _zent(o, sz):
        pltpu.sync_copy(zbuf_v.at[pl.ds(0, sz)], s_sh.at[pl.ds(o, sz)])
    _ent_copy(s, _zent)
    plsc.subcore_barrier()

    def body(ci, _):
        off = base + ci * CH
        pltpu.sync_copy(src.at[pl.ds(off, CH)], gidx_v)
        pltpu.sync_copy(et.at[pl.ds(off, CH)], eidx_v)
        pltpu.sync_copy(dst.at[pl.ds(off, CH)], didx_v)
        pltpu.async_copy(table.at[gidx_v], rows_v, sem).wait()
        pltpu.sync_copy(rows_v, seg_sh.at[eidx_v], add=True)
        pltpu.sync_copy(ones_v, cnt_sh.at[eidx_v], add=True)
        pltpu.sync_copy(rows_v, s_sh.at[didx_v], add=True)
        return 0
    lax.fori_loop(0, NCHUNK, body, 0)
    plsc.subcore_barrier()

    @pl.when(s < R2 // _REL_PER_SUB)
    def _orel():
        pltpu.sync_copy(seg_sh.at[pl.ds(s * _REL_PER_SUB, _REL_PER_SUB)],
                        seg_out.at[c, pl.ds(s * _REL_PER_SUB, _REL_PER_SUB)])
        pltpu.sync_copy(cnt_sh.at[pl.ds(s * _REL_PER_SUB, _REL_PER_SUB)],
                        cnt_out.at[c, pl.ds(s * _REL_PER_SUB, _REL_PER_SUB)])

    def _oent(o, sz):
        pltpu.sync_copy(s_sh.at[pl.ds(o, sz)], s_out.at[c, pl.ds(o, sz)])
    _ent_copy(s, _oent)


@functools.lru_cache(maxsize=None)
def _fused_sweep_kernel():
    return pl.kernel(
        _fused_sweep_body,
        out_type=(
            jax.ShapeDtypeStruct((NC, R2, H), jnp.float32),
            jax.ShapeDtypeStruct((NC, R2, H), jnp.float32),
            jax.ShapeDtypeStruct((NC, NE, H), jnp.float32),
        ),
        mesh=_sc_mesh(),
        scratch_types=[
            pltpu.VMEM((CH,), jnp.int32),
            pltpu.VMEM((CH,), jnp.int32),
            pltpu.VMEM((CH,), jnp.int32),
            pltpu.VMEM((CH, H), jnp.float32),
            pltpu.VMEM((CH, H), jnp.float32),
            pltpu.VMEM((ZB, H), jnp.float32),
            pltpu.VMEM_SHARED((R2, H), jnp.float32),
            pltpu.VMEM_SHARED((R2, H), jnp.float32),
            pltpu.VMEM_SHARED((NE, H), jnp.float32),
            pltpu.SemaphoreType.DMA,
        ],
    )


def _fused_sweep(table, src, et, dst):
    return _fused_sweep_kernel()(table, src, et, dst)


def _rel_sweep_body(table, et, dst, r_out,
                    eidx_v, didx_v, rows_v, zbuf_v, acc_sh, sem):
    c = lax.axis_index("c")
    s = lax.axis_index("s")
    wid = c * NS + s
    base = wid * EW

    _zero_vmem(zbuf_v, ZB, H)

    def _zent(o, sz):
        pltpu.sync_copy(zbuf_v.at[pl.ds(0, sz)], acc_sh.at[pl.ds(o, sz)])
    _ent_copy(s, _zent)
    plsc.subcore_barrier()

    def body(ci, _):
        off = base + ci * CH
        pltpu.sync_copy(et.at[pl.ds(off, CH)], eidx_v)
        pltpu.sync_copy(dst.at[pl.ds(off, CH)], didx_v)
        pltpu.async_copy(table.at[eidx_v], rows_v, sem).wait()
        pltpu.sync_copy(rows_v, acc_sh.at[didx_v], add=True)
        return 0
    lax.fori_loop(0, NCHUNK, body, 0)
    plsc.subcore_barrier()

    def _oent(o, sz):
        pltpu.sync_copy(acc_sh.at[pl.ds(o, sz)], r_out.at[c, pl.ds(o, sz)])
    _ent_copy(s, _oent)


@functools.lru_cache(maxsize=None)
def _rel_sweep_kernel():
    return pl.kernel(
        _rel_sweep_body,
        out_type=jax.ShapeDtypeStruct((NC, NE, H), jnp.float32),
        mesh=_sc_mesh(),
        scratch_types=[
            pltpu.VMEM((CH,), jnp.int32),
            pltpu.VMEM((CH,), jnp.int32),
            pltpu.VMEM((CH, H), jnp.float32),
            pltpu.VMEM((ZB, H), jnp.float32),
            pltpu.VMEM_SHARED((NE, H), jnp.float32),
            pltpu.SemaphoreType.DMA,
        ],
    )


def _rel_sweep(table, et, dst):
    return _rel_sweep_kernel()(table, et, dst)


def _l2norm_body(x_ref, o_ref):
    x = x_ref[...]
    n = jnp.sqrt(jnp.sum(x * x, axis=1, keepdims=True))
    o_ref[...] = x / jnp.clip(n, 1e-12, None)


def _l2norm(x):
    bn = 1000
    return pl.pallas_call(
        _l2norm_body,
        grid=(NE // bn,),
        in_specs=[pl.BlockSpec((bn, H), lambda i: (i, 0))],
        out_specs=pl.BlockSpec((bn, H), lambda i: (i, 0)),
        out_shape=jax.ShapeDtypeStruct((NE, H), jnp.float32),
    )(x)


def _dotT(a, b):
    return lax.dot_general(a, b, (((1,), (1,)), ((), ())),
                           precision=lax.Precision.HIGHEST,
                           preferred_element_type=jnp.float32)


def _dot(a, b):
    return lax.dot_general(a, b, (((1,), (0,)), ((), ())),
                           precision=lax.Precision.HIGHEST,
                           preferred_element_type=jnp.float32)


def _gru_body(seg_ref, cnt_ref, emb_ref, h0_ref, wih_ref, whh_ref,
              bih_ref, bhh_ref, wn_ref, h0o_ref, h0w_ref):
    seg = seg_ref[0] + seg_ref[1]
    cnt = cnt_ref[0, :, 0:1] + cnt_ref[1, :, 0:1]
    x_mean = seg / jnp.maximum(cnt, 1.0)
    emb = emb_ref[...]
    h0 = h0_ref[...]
    x = jnp.concatenate([emb, x_mean], axis=1)
    gi = _dotT(x, wih_ref[...]) + bih_ref[...]
    gh = _dotT(h0, whh_ref[...]) + bhh_ref[...]
    i_r, i_z, i_n = gi[:, :H], gi[:, H:2 * H], gi[:, 2 * H:]
    h_r, h_z, h_n = gh[:, :H], gh[:, H:2 * H], gh[:, 2 * H:]
    r = jax.nn.sigmoid(i_r + h_r)
    z = jax.nn.sigmoid(i_z + h_z)
    n = jnp.tanh(i_n + r * h_n)
    h0n = (1.0 - z) * n + z * h0
    h0o_ref[...] = h0n
    h0w_ref[...] = _dot(h0n, wn_ref[...])


def _gru_step(seg_p, cnt_p, emb_rel, h0, w_ih, w_hh, b_ih, b_hh, w_n):
    return pl.pallas_call(
        _gru_body,
        out_shape=(
            jax.ShapeDtypeStruct((R2, H), jnp.float32),
            jax.ShapeDtypeStruct((R2, H), jnp.float32),
        ),
    )(seg_p, cnt_p, emb_rel, h0, w_ih, w_hh, b_ih, b_hh, w_n)


def _update_body(h_ref, s_ref, r_ref, wn_ref, lw_ref, w1_ref, o_ref):
    h = h_ref[...]
    S = s_ref[0] + s_ref[1]
    R = r_ref[0] + r_ref[1]
    agg = _dot(S, wn_ref[...]) + R
    loop = _dot(h, lw_ref[...])
    cur = agg + loop
    cur = jnp.where(cur >= 0, cur, cur * NEG_SLOPE)
    n = jnp.sqrt(jnp.sum(cur * cur, axis=1, keepdims=True))
    cur = cur / jnp.clip(n, 1e-12, None)
    tw = jax.nn.sigmoid(_dot(h, w1_ref[...]))
    o_ref[...] = tw * cur + (1.0 - tw) * h


def _update_step(h, s_p, r_p, w_n, loop_w, w1):
    bn = 1000
    return pl.pallas_call(
        _update_body,
        grid=(NE // bn,),
        in_specs=[
            pl.BlockSpec((bn, H), lambda i: (i, 0)),
            pl.BlockSpec((NC, bn, H), lambda i: (0, i, 0)),
            pl.BlockSpec((NC, bn, H), lambda i: (0, i, 0)),
            pl.BlockSpec((H, H), lambda i: (0, 0)),
            pl.BlockSpec((H, H), lambda i: (0, 0)),
            pl.BlockSpec((H, H), lambda i: (0, 0)),
        ],
        out_specs=pl.BlockSpec((bn, H), lambda i: (i, 0)),
        out_shape=jax.ShapeDtypeStruct((NE, H), jnp.float32),
    )(h, s_p, r_p, w_n, loop_w, w1)


def kernel(edge_index, edge_type, use_cuda, dynamic_emb, emb_rel, w1,
           weight_neighbor, loop_weight, w_ih, w_hh, b_ih, b_hh):
    del use_cuda
    h = _l2norm(dynamic_emb)
    h0 = emb_rel
    b_ih2 = b_ih.reshape(1, 3 * H)
    b_hh2 = b_hh.reshape(1, 3 * H)
    outs = []
    for t in range(TSTEPS):
        src = edge_index[t, 0]
        dst = edge_index[t, 1]
        et = edge_type[t]
        seg_p, cnt_p, s_p = _fused_sweep(h, src, et, dst)
        h0, h0w = _gru_step(seg_p, cnt_p, emb_rel, h0, w_ih, w_hh,
                            b_ih2, b_hh2, weight_neighbor)
        r_p = _rel_sweep(h0w, et, dst)
        h = _update_step(h, s_p, r_p, weight_neighbor, loop_weight, w1)
        outs.append(h)
    return jnp.stack(outs, axis=0)


# speedup vs baseline: 3.9416x; 1.0682x over previous
import functools

import jax
import jax.numpy as jnp
from jax import lax
from jax.experimental import pallas as pl
from jax.experimental.pallas import tpu as pltpu
from jax.experimental.pallas import tpu_sc as plsc

NE = 10000
H = 128
EDGES = 320000
R2 = 400
TSTEPS = 3

NC = 2
NS = 16
NW = NC * NS
EW = EDGES // NW
CH = 80
NCHUNK = EW // CH
ZB = 104

_REL_PER_SUB = 40
_ENT_PER_SUB = 624

NEG_SLOPE = (1.0 / 8.0 + 1.0 / 3.0) / 2.0


def _sc_mesh():
    return plsc.VectorSubcoreMesh(
        core_axis_name="c", subcore_axis_name="s", num_cores=NC, num_subcores=NS
    )


def _zero_vmem(ref, nrows, ncols):
    def row(i, _):
        def col(j, __):
            ref[i, pl.ds(j * 16, 16)] = jnp.zeros((16,), jnp.float32)
            return 0
        lax.fori_loop(0, ncols // 16, col, 0)
        return 0
    lax.fori_loop(0, nrows, row, 0)


def _ent_copy(s, fn):
    for r in range(_ENT_PER_SUB // ZB):
        fn(s * _ENT_PER_SUB + r * ZB, ZB)

    @pl.when(s == NS - 1)
    def _tail():
        fn(NS * _ENT_PER_SUB, NE - NS * _ENT_PER_SUB)


def _fused_sweep_body(table, src, et, dst, seg_out, s_out,
                      gidx_v, eidx_v, didx_v, rows_v, zbuf_v,
                      seg_sh, s_sh, sem):
    c = lax.axis_index("c")
    s = lax.axis_index("s")
    wid = c * NS + s
    base = wid * EW

    _zero_vmem(zbuf_v, ZB, H)

    @pl.when(s < R2 // _REL_PER_SUB)
    def _zrel():
        pltpu.sync_copy(zbuf_v.at[pl.ds(0, _REL_PER_SUB)],
                        seg_sh.at[pl.ds(s * _REL_PER_SUB, _REL_PER_SUB)])

    def _zent(o, sz):
        pltpu.sync_copy(zbuf_v.at[pl.ds(0, sz)], s_sh.at[pl.ds(o, sz)])
    _ent_copy(s, _zent)
    plsc.subcore_barrier()

    def body(ci, _):
        off = base + ci * CH
        pltpu.sync_copy(src.at[pl.ds(off, CH)], gidx_v)
        pltpu.sync_copy(et.at[pl.ds(off, CH)], eidx_v)
        pltpu.sync_copy(dst.at[pl.ds(off, CH)], didx_v)
        pltpu.async_copy(table.at[gidx_v], rows_v, sem).wait()
        pltpu.sync_copy(rows_v, seg_sh.at[eidx_v], add=True)
        pltpu.sync_copy(rows_v, s_sh.at[didx_v], add=True)
        return 0
    lax.fori_loop(0, NCHUNK, body, 0)
    plsc.subcore_barrier()

    @pl.when(s < R2 // _REL_PER_SUB)
    def _orel():
        pltpu.sync_copy(seg_sh.at[pl.ds(s * _REL_PER_SUB, _REL_PER_SUB)],
                        seg_out.at[c, pl.ds(s * _REL_PER_SUB, _REL_PER_SUB)])

    def _oent(o, sz):
        pltpu.sync_copy(s_sh.at[pl.ds(o, sz)], s_out.at[c, pl.ds(o, sz)])
    _ent_copy(s, _oent)


@functools.lru_cache(maxsize=None)
def _fused_sweep_kernel():
    return pl.kernel(
        _fused_sweep_body,
        out_type=(
            jax.ShapeDtypeStruct((NC, R2, H), jnp.float32),
            jax.ShapeDtypeStruct((NC, NE, H), jnp.float32),
        ),
        mesh=_sc_mesh(),
        scratch_types=[
            pltpu.VMEM((CH,), jnp.int32),
            pltpu.VMEM((CH,), jnp.int32),
            pltpu.VMEM((CH,), jnp.int32),
            pltpu.VMEM((CH, H), jnp.float32),
            pltpu.VMEM((ZB, H), jnp.float32),
            pltpu.VMEM_SHARED((R2, H), jnp.float32),
            pltpu.VMEM_SHARED((NE, H), jnp.float32),
            pltpu.SemaphoreType.DMA,
        ],
    )


def _fused_sweep(table, src, et, dst):
    return _fused_sweep_kernel()(table, src, et, dst)


def _rel_sweep_body(table, et, dst, r_out,
                    eidx_v, didx_v, rows_v, zbuf_v, acc_sh, sem):
    c = lax.axis_index("c")
    s = lax.axis_index("s")
    wid = c * NS + s
    base = wid * EW

    _zero_vmem(zbuf_v, ZB, H)

    def _zent(o, sz):
        pltpu.sync_copy(zbuf_v.at[pl.ds(0, sz)], acc_sh.at[pl.ds(o, sz)])
    _ent_copy(s, _zent)
    plsc.subcore_barrier()

    def body(ci, _):
        off = base + ci * CH
        pltpu.sync_copy(et.at[pl.ds(off, CH)], eidx_v)
        pltpu.sync_copy(dst.at[pl.ds(off, CH)], didx_v)
        pltpu.async_copy(table.at[eidx_v], rows_v, sem).wait()
        pltpu.sync_copy(rows_v, acc_sh.at[didx_v], add=True)
        return 0
    lax.fori_loop(0, NCHUNK, body, 0)
    plsc.subcore_barrier()

    def _oent(o, sz):
        pltpu.sync_copy(acc_sh.at[pl.ds(o, sz)], r_out.at[c, pl.ds(o, sz)])
    _ent_copy(s, _oent)


@functools.lru_cache(maxsize=None)
def _rel_sweep_kernel():
    return pl.kernel(
        _rel_sweep_body,
        out_type=jax.ShapeDtypeStruct((NC, NE, H), jnp.float32),
        mesh=_sc_mesh(),
        scratch_types=[
            pltpu.VMEM((CH,), jnp.int32),
            pltpu.VMEM((CH,), jnp.int32),
            pltpu.VMEM((CH, H), jnp.float32),
            pltpu.VMEM((ZB, H), jnp.float32),
            pltpu.VMEM_SHARED((NE, H), jnp.float32),
            pltpu.SemaphoreType.DMA,
        ],
    )


def _rel_sweep(table, et, dst):
    return _rel_sweep_kernel()(table, et, dst)


_CNT_BLK = 2000
_CNT_NBLK = EDGES // _CNT_BLK


def _cnt_body(et_ref, o_ref):
    i = pl.program_id(0)
    et = et_ref[0, 0, :]
    onehot = (et[:, None] ==
              lax.broadcasted_iota(jnp.int32, (1, R2), 1)).astype(jnp.float32)
    c = jnp.sum(onehot, axis=0, keepdims=True)
    val = jnp.broadcast_to(c[:, :, None], (1, R2, H))

    @pl.when(i % _CNT_NBLK == 0)
    def _init():
        o_ref[...] = jnp.zeros_like(o_ref)

    o_ref[...] = o_ref[...] + val


def _cnt_tc(et2d):
    return pl.pallas_call(
        _cnt_body,
        grid=(TSTEPS * _CNT_NBLK,),
        in_specs=[pl.BlockSpec((1, 1, _CNT_BLK), lambda i: (i, 0, 0))],
        out_specs=pl.BlockSpec((1, R2, H), lambda i: (i // _CNT_NBLK, 0, 0)),
        out_shape=jax.ShapeDtypeStruct((TSTEPS, R2, H), jnp.float32),
    )(et2d)

def _l2norm_body(x_ref, o_ref):
    x = x_ref[...]
    n = jnp.sqrt(jnp.sum(x * x, axis=1, keepdims=True))
    o_ref[...] = x / jnp.clip(n, 1e-12, None)


def _l2norm(x):
    bn = 1000
    return pl.pallas_call(
        _l2norm_body,
        grid=(NE // bn,),
        in_specs=[pl.BlockSpec((bn, H), lambda i: (i, 0))],
        out_specs=pl.BlockSpec((bn, H), lambda i: (i, 0)),
        out_shape=jax.ShapeDtypeStruct((NE, H), jnp.float32),
    )(x)


def _dotT(a, b):
    return lax.dot_general(a, b, (((1,), (1,)), ((), ())),
                           precision=lax.Precision.HIGHEST,
                           preferred_element_type=jnp.float32)


def _dot(a, b):
    return lax.dot_general(a, b, (((1,), (0,)), ((), ())),
                           precision=lax.Precision.HIGHEST,
                           preferred_element_type=jnp.float32)


def _gru_body(seg_ref, cnt_ref, emb_ref, h0_ref, wih_ref, whh_ref,
              bih_ref, bhh_ref, wn_ref, h0o_ref, h0w_ref):
    seg = seg_ref[0] + seg_ref[1]
    x_mean = seg / jnp.maximum(cnt_ref[...], 1.0)
    emb = emb_ref[...]
    h0 = h0_ref[...]
    x = jnp.concatenate([emb, x_mean], axis=1)
    gi = _dotT(x, wih_ref[...]) + bih_ref[...]
    gh = _dotT(h0, whh_ref[...]) + bhh_ref[...]
    i_r, i_z, i_n = gi[:, :H], gi[:, H:2 * H], gi[:, 2 * H:]
    h_r, h_z, h_n = gh[:, :H], gh[:, H:2 * H], gh[:, 2 * H:]
    r = jax.nn.sigmoid(i_r + h_r)
    z = jax.nn.sigmoid(i_z + h_z)
    n = jnp.tanh(i_n + r * h_n)
    h0n = (1.0 - z) * n + z * h0
    h0o_ref[...] = h0n
    h0w_ref[...] = _dot(h0n, wn_ref[...])


def _gru_step(seg_p, cnt_t, emb_rel, h0, w_ih, w_hh, b_ih, b_hh, w_n):
    return pl.pallas_call(
        _gru_body,
        out_shape=(
            jax.ShapeDtypeStruct((R2, H), jnp.float32),
            jax.ShapeDtypeStruct((R2, H), jnp.float32),
        ),
    )(seg_p, cnt_t, emb_rel, h0, w_ih, w_hh, b_ih, b_hh, w_n)


def _update_body(h_ref, s_ref, r_ref, wn_ref, lw_ref, w1_ref, o_ref):
    h = h_ref[...]
    S = s_ref[0] + s_ref[1]
    R = r_ref[0] + r_ref[1]
    agg = _dot(S, wn_ref[...]) + R
    loop = _dot(h, lw_ref[...])
    cur = agg + loop
    cur = jnp.where(cur >= 0, cur, cur * NEG_SLOPE)
    n = jnp.sqrt(jnp.sum(cur * cur, axis=1, keepdims=True))
    cur = cur / jnp.clip(n, 1e-12, None)
    tw = jax.nn.sigmoid(_dot(h, w1_ref[...]))
    o_ref[...] = tw * cur + (1.0 - tw) * h


def _update_step(h, s_p, r_p, w_n, loop_w, w1):
    bn = 1000
    return pl.pallas_call(
        _update_body,
        grid=(NE // bn,),
        in_specs=[
            pl.BlockSpec((bn, H), lambda i: (i, 0)),
            pl.BlockSpec((NC, bn, H), lambda i: (0, i, 0)),
            pl.BlockSpec((NC, bn, H), lambda i: (0, i, 0)),
            pl.BlockSpec((H, H), lambda i: (0, 0)),
            pl.BlockSpec((H, H), lambda i: (0, 0)),
            pl.BlockSpec((H, H), lambda i: (0, 0)),
        ],
        out_specs=pl.BlockSpec((bn, H), lambda i: (i, 0)),
        out_shape=jax.ShapeDtypeStruct((NE, H), jnp.float32),
    )(h, s_p, r_p, w_n, loop_w, w1)


def kernel(edge_index, edge_type, use_cuda, dynamic_emb, emb_rel, w1,
           weight_neighbor, loop_weight, w_ih, w_hh, b_ih, b_hh):
    del use_cuda
    h = _l2norm(dynamic_emb)
    h0 = emb_rel
    b_ih2 = b_ih.reshape(1, 3 * H)
    b_hh2 = b_hh.reshape(1, 3 * H)
    cnts = _cnt_tc(edge_type.reshape(TSTEPS * _CNT_NBLK, 1, _CNT_BLK))
    outs = []
    for t in range(TSTEPS):
        src = edge_index[t, 0]
        dst = edge_index[t, 1]
        et = edge_type[t]
        seg_p, s_p = _fused_sweep(h, src, et, dst)
        h0, h0w = _gru_step(seg_p, cnts[t], emb_rel, h0, w_ih, w_hh,
                            b_ih2, b_hh2, weight_neighbor)
        r_p = _rel_sweep(h0w, et, dst)
        h = _update_step(h, s_p, r_p, weight_neighbor, loop_weight, w1)
        outs.append(h)
    return jnp.stack(outs, axis=0)


# speedup vs baseline: 6.5523x; 1.6623x over previous
import functools

import jax
import jax.numpy as jnp
from jax import lax
from jax.experimental import pallas as pl
from jax.experimental.pallas import tpu as pltpu
from jax.experimental.pallas import tpu_sc as plsc

NE = 10000
H = 128
EDGES = 320000
R2 = 400
TSTEPS = 3

NC = 2
NS = 16
NW = NC * NS
EW = EDGES // NW
CH = 80
NCHUNK = EW // CH
ZB = 48
NB = 5
NR = 3

_REL_PER_SUB = 40
_ENT_PER_SUB = 624

NEG_SLOPE = (1.0 / 8.0 + 1.0 / 3.0) / 2.0


def _sc_mesh():
    return plsc.VectorSubcoreMesh(
        core_axis_name="c", subcore_axis_name="s", num_cores=NC, num_subcores=NS
    )


def _zero_vmem(ref, nrows, ncols):
    def row(i, _):
        def col(j, __):
            ref[i, pl.ds(j * 16, 16)] = jnp.zeros((16,), jnp.float32)
            return 0
        lax.fori_loop(0, ncols // 16, col, 0)
        return 0
    lax.fori_loop(0, nrows, row, 0)


def _ent_copy(s, fn):
    for r in range(_ENT_PER_SUB // ZB):
        fn(s * _ENT_PER_SUB + r * ZB, ZB)

    @pl.when(s == NS - 1)
    def _tail():
        fn(NS * _ENT_PER_SUB, NE - NS * _ENT_PER_SUB)


def _fused_sweep_body(table, src, et, dst, seg_out, s_out, *scr):
    gidx_v = scr[0:NB]
    eidx_v = scr[NB:2 * NB]
    didx_v = scr[2 * NB:3 * NB]
    rows_v = scr[3 * NB:3 * NB + NR]
    zbuf_v, seg_sh, s_sh, isem, gsem, ssem = scr[3 * NB + NR:]
    c = lax.axis_index("c")
    s = lax.axis_index("s")
    wid = c * NS + s
    base = wid * EW

    _zero_vmem(zbuf_v, ZB, H)

    @pl.when(s < R2 // _REL_PER_SUB)
    def _zrel():
        pltpu.sync_copy(zbuf_v.at[pl.ds(0, _REL_PER_SUB)],
                        seg_sh.at[pl.ds(s * _REL_PER_SUB, _REL_PER_SUB)])

    def _zent(o, sz):
        pltpu.sync_copy(zbuf_v.at[pl.ds(0, sz)], s_sh.at[pl.ds(o, sz)])
    _ent_copy(s, _zent)
    plsc.subcore_barrier()

    def group(g, _):
        gb = base + g * (NB * CH)
        idesc = []
        for b in range(NB):
            off = gb + b * CH
            idesc.append([
                pltpu.async_copy(src.at[pl.ds(off, CH)], gidx_v[b], isem),
                pltpu.async_copy(et.at[pl.ds(off, CH)], eidx_v[b], isem),
                pltpu.async_copy(dst.at[pl.ds(off, CH)], didx_v[b], isem),
            ])
        gdesc = [None] * NB
        sdesc = [None] * NB
        for b in range(NB):
            rb = b % NR
            if b >= NR:
                for d in sdesc[b - NR]:
                    d.wait()
            for d in idesc[b]:
                d.wait()
            gdesc[b] = pltpu.async_copy(table.at[gidx_v[b]], rows_v[rb], gsem)
            if b >= 1:
                gdesc[b - 1].wait()
                pb = (b - 1) % NR
                sdesc[b - 1] = [
                    pltpu.async_copy(rows_v[pb], seg_sh.at[eidx_v[b - 1]],
                                     ssem, add=True),
                    pltpu.async_copy(rows_v[pb], s_sh.at[didx_v[b - 1]],
                                     ssem, add=True),
                ]
        gdesc[NB - 1].wait()
        lb = (NB - 1) % NR
        sdesc[NB - 1] = [
            pltpu.async_copy(rows_v[lb], seg_sh.at[eidx_v[NB - 1]],
                             ssem, add=True),
            pltpu.async_copy(rows_v[lb], s_sh.at[didx_v[NB - 1]],
                             ssem, add=True),
        ]
        for b in range(max(NB - NR, 0), NB):
            for d in sdesc[b]:
                d.wait()
        return 0
    lax.fori_loop(0, NCHUNK // NB, group, 0)
    plsc.subcore_barrier()

    @pl.when(s < R2 // _REL_PER_SUB)
    def _orel():
        pltpu.sync_copy(seg_sh.at[pl.ds(s * _REL_PER_SUB, _REL_PER_SUB)],
                        seg_out.at[c, pl.ds(s * _REL_PER_SUB, _REL_PER_SUB)])

    def _oent(o, sz):
        pltpu.sync_copy(s_sh.at[pl.ds(o, sz)], s_out.at[c, pl.ds(o, sz)])
    _ent_copy(s, _oent)


@functools.lru_cache(maxsize=None)
def _fused_sweep_kernel():
    return pl.kernel(
        _fused_sweep_body,
        out_type=(
            jax.ShapeDtypeStruct((NC, R2, H), jnp.float32),
            jax.ShapeDtypeStruct((NC, NE, H), jnp.float32),
        ),
        mesh=_sc_mesh(),
        scratch_types=(
            [pltpu.VMEM((CH,), jnp.int32)] * (3 * NB)
            + [pltpu.VMEM((CH, H), jnp.float32)] * NR
            + [
                pltpu.VMEM((ZB, H), jnp.float32),
                pltpu.VMEM_SHARED((R2, H), jnp.float32),
                pltpu.VMEM_SHARED((NE, H), jnp.float32),
                pltpu.SemaphoreType.DMA,
                pltpu.SemaphoreType.DMA,
                pltpu.SemaphoreType.DMA,
            ]
        ),
    )


def _fused_sweep(table, src, et, dst):
    return _fused_sweep_kernel()(table, src, et, dst)


def _rel_sweep_body(table, et, dst, r_out, *scr):
    eidx_v = scr[0:NB]
    didx_v = scr[NB:2 * NB]
    rows_v = scr[2 * NB:2 * NB + NR]
    zbuf_v, acc_sh, isem, gsem, ssem = scr[2 * NB + NR:]
    c = lax.axis_index("c")
    s = lax.axis_index("s")
    wid = c * NS + s
    base = wid * EW

    _zero_vmem(zbuf_v, ZB, H)

    def _zent(o, sz):
        pltpu.sync_copy(zbuf_v.at[pl.ds(0, sz)], acc_sh.at[pl.ds(o, sz)])
    _ent_copy(s, _zent)
    plsc.subcore_barrier()

    def group(g, _):
        gb = base + g * (NB * CH)
        idesc = []
        for b in range(NB):
            off = gb + b * CH
            idesc.append([
                pltpu.async_copy(et.at[pl.ds(off, CH)], eidx_v[b], isem),
                pltpu.async_copy(dst.at[pl.ds(off, CH)], didx_v[b], isem),
            ])
        gdesc = [None] * NB
        sdesc = [None] * NB
        for b in range(NB):
            rb = b % NR
            if b >= NR:
                sdesc[b - NR].wait()
            for d in idesc[b]:
                d.wait()
            gdesc[b] = pltpu.async_copy(table.at[eidx_v[b]], rows_v[rb], gsem)
            if b >= 1:
                gdesc[b - 1].wait()
                sdesc[b - 1] = pltpu.async_copy(
                    rows_v[(b - 1) % NR], acc_sh.at[didx_v[b - 1]],
                    ssem, add=True)
        gdesc[NB - 1].wait()
        sdesc[NB - 1] = pltpu.async_copy(
            rows_v[(NB - 1) % NR], acc_sh.at[didx_v[NB - 1]], ssem, add=True)
        for b in range(max(NB - NR, 0), NB):
            sdesc[b].wait()
        return 0
    lax.fori_loop(0, NCHUNK // NB, group, 0)
    plsc.subcore_barrier()

    def _oent(o, sz):
        pltpu.sync_copy(acc_sh.at[pl.ds(o, sz)], r_out.at[c, pl.ds(o, sz)])
    _ent_copy(s, _oent)


@functools.lru_cache(maxsize=None)
def _rel_sweep_kernel():
    return pl.kernel(
        _rel_sweep_body,
        out_type=jax.ShapeDtypeStruct((NC, NE, H), jnp.float32),
        mesh=_sc_mesh(),
        scratch_types=(
            [pltpu.VMEM((CH,), jnp.int32)] * (2 * NB)
            + [pltpu.VMEM((CH, H), jnp.float32)] * NR
            + [
                pltpu.VMEM((ZB, H), jnp.float32),
                pltpu.VMEM_SHARED((NE, H), jnp.float32),
                pltpu.SemaphoreType.DMA,
                pltpu.SemaphoreType.DMA,
                pltpu.SemaphoreType.DMA,
            ]
        ),
    )


def _rel_sweep(table, et, dst):
    return _rel_sweep_kernel()(table, et, dst)


_CNT_BLK = 2000
_CNT_NBLK = EDGES // _CNT_BLK


def _cnt_body(et_ref, o_ref):
    i = pl.program_id(0)
    et = et_ref[0, 0, :]
    onehot = (et[:, None] ==
              lax.broadcasted_iota(jnp.int32, (1, R2), 1)).astype(jnp.float32)
    c = jnp.sum(onehot, axis=0, keepdims=True)
    val = jnp.broadcast_to(c[:, :, None], (1, R2, H))

    @pl.when(i % _CNT_NBLK == 0)
    def _init():
        o_ref[...] = jnp.zeros_like(o_ref)

    o_ref[...] = o_ref[...] + val


def _cnt_tc(et2d):
    return pl.pallas_call(
        _cnt_body,
        grid=(TSTEPS * _CNT_NBLK,),
        in_specs=[pl.BlockSpec((1, 1, _CNT_BLK), lambda i: (i, 0, 0))],
        out_specs=pl.BlockSpec((1, R2, H), lambda i: (i // _CNT_NBLK, 0, 0)),
        out_shape=jax.ShapeDtypeStruct((TSTEPS, R2, H), jnp.float32),
    )(et2d)

def _l2norm_body(x_ref, o_ref):
    x = x_ref[...]
    n = jnp.sqrt(jnp.sum(x * x, axis=1, keepdims=True))
    o_ref[...] = x / jnp.clip(n, 1e-12, None)


def _l2norm(x):
    bn = 1000
    return pl.pallas_call(
        _l2norm_body,
        grid=(NE // bn,),
        in_specs=[pl.BlockSpec((bn, H), lambda i: (i, 0))],
        out_specs=pl.BlockSpec((bn, H), lambda i: (i, 0)),
        out_shape=jax.ShapeDtypeStruct((NE, H), jnp.float32),
    )(x)


def _dotT(a, b):
    return lax.dot_general(a, b, (((1,), (1,)), ((), ())),
                           precision=lax.Precision.HIGHEST,
                           preferred_element_type=jnp.float32)


def _dot(a, b):
    return lax.dot_general(a, b, (((1,), (0,)), ((), ())),
                           precision=lax.Precision.HIGHEST,
                           preferred_element_type=jnp.float32)


def _gru_body(seg_ref, cnt_ref, emb_ref, h0_ref, wih_ref, whh_ref,
              bih_ref, bhh_ref, wn_ref, h0o_ref, h0w_ref):
    seg = seg_ref[0] + seg_ref[1]
    x_mean = seg / jnp.maximum(cnt_ref[...], 1.0)
    emb = emb_ref[...]
    h0 = h0_ref[...]
    x = jnp.concatenate([emb, x_mean], axis=1)
    gi = _dotT(x, wih_ref[...]) + bih_ref[...]
    gh = _dotT(h0, whh_ref[...]) + bhh_ref[...]
    i_r, i_z, i_n = gi[:, :H], gi[:, H:2 * H], gi[:, 2 * H:]
    h_r, h_z, h_n = gh[:, :H], gh[:, H:2 * H], gh[:, 2 * H:]
    r = jax.nn.sigmoid(i_r + h_r)
    z = jax.nn.sigmoid(i_z + h_z)
    n = jnp.tanh(i_n + r * h_n)
    h0n = (1.0 - z) * n + z * h0
    h0o_ref[...] = h0n
    h0w_ref[...] = _dot(h0n, wn_ref[...])


def _gru_step(seg_p, cnt_t, emb_rel, h0, w_ih, w_hh, b_ih, b_hh, w_n):
    return pl.pallas_call(
        _gru_body,
        out_shape=(
            jax.ShapeDtypeStruct((R2, H), jnp.float32),
            jax.ShapeDtypeStruct((R2, H), jnp.float32),
        ),
    )(seg_p, cnt_t, emb_rel, h0, w_ih, w_hh, b_ih, b_hh, w_n)


def _update_body(h_ref, s_ref, r_ref, wn_ref, lw_ref, w1_ref, o_ref):
    h = h_ref[...]
    S = s_ref[0] + s_ref[1]
    R = r_ref[0] + r_ref[1]
    agg = _dot(S, wn_ref[...]) + R
    loop = _dot(h, lw_ref[...])
    cur = agg + loop
    cur = jnp.where(cur >= 0, cur, cur * NEG_SLOPE)
    n = jnp.sqrt(jnp.sum(cur * cur, axis=1, keepdims=True))
    cur = cur / jnp.clip(n, 1e-12, None)
    tw = jax.nn.sigmoid(_dot(h, w1_ref[...]))
    o_ref[...] = tw * cur + (1.0 - tw) * h


def _update_step(h, s_p, r_p, w_n, loop_w, w1):
    bn = 1000
    return pl.pallas_call(
        _update_body,
        grid=(NE // bn,),
        in_specs=[
            pl.BlockSpec((bn, H), lambda i: (i, 0)),
            pl.BlockSpec((NC, bn, H), lambda i: (0, i, 0)),
            pl.BlockSpec((NC, bn, H), lambda i: (0, i, 0)),
            pl.BlockSpec((H, H), lambda i: (0, 0)),
            pl.BlockSpec((H, H), lambda i: (0, 0)),
            pl.BlockSpec((H, H), lambda i: (0, 0)),
        ],
        out_specs=pl.BlockSpec((bn, H), lambda i: (i, 0)),
        out_shape=jax.ShapeDtypeStruct((NE, H), jnp.float32),
    )(h, s_p, r_p, w_n, loop_w, w1)


def kernel(edge_index, edge_type, use_cuda, dynamic_emb, emb_rel, w1,
           weight_neighbor, loop_weight, w_ih, w_hh, b_ih, b_hh):
    del use_cuda
    h = _l2norm(dynamic_emb)
    h0 = emb_rel
    b_ih2 = b_ih.reshape(1, 3 * H)
    b_hh2 = b_hh.reshape(1, 3 * H)
    cnts = _cnt_tc(edge_type.reshape(TSTEPS * _CNT_NBLK, 1, _CNT_BLK))
    outs = []
    for t in range(TSTEPS):
        src = edge_index[t, 0]
        dst = edge_index[t, 1]
        et = edge_type[t]
        seg_p, s_p = _fused_sweep(h, src, et, dst)
        h0, h0w = _gru_step(seg_p, cnts[t], emb_rel, h0, w_ih, w_hh,
                            b_ih2, b_hh2, weight_neighbor)
        r_p = _rel_sweep(h0w, et, dst)
        h = _update_step(h, s_p, r_p, weight_neighbor, loop_weight, w1)
        outs.append(h)
    return jnp.stack(outs, axis=0)


# speedup vs baseline: 9.0158x; 1.3760x over previous
import functools

import jax
import jax.numpy as jnp
from jax import lax
from jax.experimental import pallas as pl
from jax.experimental.pallas import tpu as pltpu
from jax.experimental.pallas import tpu_sc as plsc

NE = 10000
H = 128
EDGES = 320000
R2 = 400
TSTEPS = 3

NC = 2
NS = 16
NW = NC * NS
EW = EDGES // NW
CH = 80
NCHUNK = EW // CH
ZB = 48
NB = 5
NR = 3

_REL_PER_SUB = 40
_ENT_PER_SUB = 624

NEG_SLOPE = (1.0 / 8.0 + 1.0 / 3.0) / 2.0


def _sc_mesh():
    return plsc.VectorSubcoreMesh(
        core_axis_name="c", subcore_axis_name="s", num_cores=NC, num_subcores=NS
    )


def _zero_vmem(ref, nrows, ncols):
    def row(i, _):
        def col(j, __):
            ref[i, pl.ds(j * 16, 16)] = jnp.zeros((16,), jnp.float32)
            return 0
        lax.fori_loop(0, ncols // 16, col, 0)
        return 0
    lax.fori_loop(0, nrows, row, 0)


def _ent_copy(s, fn):
    for r in range(_ENT_PER_SUB // ZB):
        fn(s * _ENT_PER_SUB + r * ZB, ZB)

    @pl.when(s == NS - 1)
    def _tail():
        fn(NS * _ENT_PER_SUB, NE - NS * _ENT_PER_SUB)


def _fused_sweep_body(table, src, et, dst, seg_out, s_out, *scr):
    gidx_v = scr[0:NB]
    eidx_v = scr[NB:2 * NB]
    didx_v = scr[2 * NB:3 * NB]
    rows_v = scr[3 * NB:3 * NB + NR]
    zbuf_v, seg_sh, s_sh, isem, gsem, ssem = scr[3 * NB + NR:]
    c = lax.axis_index("c")
    s = lax.axis_index("s")
    wid = c * NS + s
    base = wid * EW

    _zero_vmem(zbuf_v, ZB, H)

    @pl.when(s < R2 // _REL_PER_SUB)
    def _zrel():
        pltpu.sync_copy(zbuf_v.at[pl.ds(0, _REL_PER_SUB)],
                        seg_sh.at[pl.ds(s * _REL_PER_SUB, _REL_PER_SUB)])

    def _zent(o, sz):
        pltpu.sync_copy(zbuf_v.at[pl.ds(0, sz)], s_sh.at[pl.ds(o, sz)])
    _ent_copy(s, _zent)
    plsc.subcore_barrier()

    def group(g, _):
        gb = base + g * (NB * CH)
        idesc = []
        for b in range(NB):
            off = gb + b * CH
            idesc.append([
                pltpu.async_copy(src.at[pl.ds(off, CH)], gidx_v[b], isem),
                pltpu.async_copy(et.at[pl.ds(off, CH)], eidx_v[b], isem),
                pltpu.async_copy(dst.at[pl.ds(off, CH)], didx_v[b], isem),
            ])
        gdesc = [None] * NB
        sdesc = [None] * NB
        for b in range(NB):
            rb = b % NR
            if b >= NR:
                for d in sdesc[b - NR]:
                    d.wait()
            for d in idesc[b]:
                d.wait()
            gdesc[b] = pltpu.async_copy(table.at[gidx_v[b]], rows_v[rb], gsem)
            if b >= 1:
                gdesc[b - 1].wait()
                pb = (b - 1) % NR
                sdesc[b - 1] = [
                    pltpu.async_copy(rows_v[pb], seg_sh.at[eidx_v[b - 1]],
                                     ssem, add=True),
                    pltpu.async_copy(rows_v[pb], s_sh.at[didx_v[b - 1]],
                                     ssem, add=True),
                ]
        gdesc[NB - 1].wait()
        lb = (NB - 1) % NR
        sdesc[NB - 1] = [
            pltpu.async_copy(rows_v[lb], seg_sh.at[eidx_v[NB - 1]],
                             ssem, add=True),
            pltpu.async_copy(rows_v[lb], s_sh.at[didx_v[NB - 1]],
                             ssem, add=True),
        ]
        for b in range(max(NB - NR, 0), NB):
            for d in sdesc[b]:
                d.wait()
        return 0
    lax.fori_loop(0, NCHUNK // NB, group, 0)
    plsc.subcore_barrier()

    @pl.when(s < R2 // _REL_PER_SUB)
    def _orel():
        pltpu.sync_copy(seg_sh.at[pl.ds(s * _REL_PER_SUB, _REL_PER_SUB)],
                        seg_out.at[c, pl.ds(s * _REL_PER_SUB, _REL_PER_SUB)])

    def _oent(o, sz):
        pltpu.sync_copy(s_sh.at[pl.ds(o, sz)], s_out.at[c, pl.ds(o, sz)])
    _ent_copy(s, _oent)


@functools.lru_cache(maxsize=None)
def _fused_sweep_kernel():
    return pl.kernel(
        _fused_sweep_body,
        out_type=(
            jax.ShapeDtypeStruct((NC, R2, H), jnp.float32),
            jax.ShapeDtypeStruct((NC, NE, H), jnp.float32),
        ),
        mesh=_sc_mesh(),
        scratch_types=(
            [pltpu.VMEM((CH,), jnp.int32)] * (3 * NB)
            + [pltpu.VMEM((CH, H), jnp.float32)] * NR
            + [
                pltpu.VMEM((ZB, H), jnp.float32),
                pltpu.VMEM_SHARED((R2, H), jnp.float32),
                pltpu.VMEM_SHARED((NE, H), jnp.float32),
                pltpu.SemaphoreType.DMA,
                pltpu.SemaphoreType.DMA,
                pltpu.SemaphoreType.DMA,
            ]
        ),
    )


def _fused_sweep(table, src, et, dst):
    return _fused_sweep_kernel()(table, src, et, dst)


_CB_CH = 80
_CB_EPT = EDGES // NS
_CB_GRP = 10
_CB_NGRP = _CB_EPT // (_CB_CH * _CB_GRP)
_CROWS = NE // NC
_CFLAT = _CROWS * R2
_CSUB = 125056
_COUT = _CSUB * NS
_CDUM = _COUT
_CZB = 2048
_CACC = _CZB * 62 * NS


def _cbuild_body(dstf, etf, *args):
    outs = args[:TSTEPS * NC]
    (didx0, didx1, eidx0, eidx1, fidx0, fidx1, ones_v, zbuf_v,
     accf, isem, ssem, zsem) = args[TSTEPS * NC:]
    didx_v = (didx0, didx1)
    eidx_v = (eidx0, eidx1)
    fidx_v = (fidx0, fidx1)
    c = lax.axis_index("c")
    s = lax.axis_index("s")
    lo = c * _CROWS
    hi = lo + _CROWS

    def f1(k, _):
        ones_v[pl.ds(k * 16, 16)] = jnp.ones((16,), jnp.float32)
        return 0
    lax.fori_loop(0, _CB_CH // 16, f1, 0)

    def fz(k, _):
        zbuf_v[pl.ds(k * 16, 16)] = jnp.zeros((16,), jnp.float32)
        return 0
    lax.fori_loop(0, _CZB // 16, fz, 0)

    for t in range(TSTEPS):
        zd = []
        for r in range(62):
            o = (s * 62 + r) * _CZB
            zd.append(pltpu.async_copy(zbuf_v, accf.at[pl.ds(o, _CZB)], zsem))
        for d in zd:
            d.wait()
        plsc.subcore_barrier()

        toff = t * EDGES + s * _CB_EPT

        def group(g, _, toff=toff):
            tbase = toff + g * (_CB_GRP * _CB_CH)
            idesc = [None] * _CB_GRP
            sdesc = [None] * _CB_GRP

            def issue_idx(j):
                off = tbase + j * _CB_CH
                b = j % 2
                return [
                    pltpu.async_copy(dstf.at[pl.ds(off, _CB_CH)],
                                     didx_v[b], isem),
                    pltpu.async_copy(etf.at[pl.ds(off, _CB_CH)],
                                     eidx_v[b], isem),
                ]
            idesc[0] = issue_idx(0)
            idesc[1] = issue_idx(1)
            for j in range(_CB_GRP):
                b = j % 2
                if j >= 2:
                    sdesc[j - 2].wait()
                for d in idesc[j]:
                    d.wait()
                for k in range(_CB_CH // 16):
                    d16 = didx_v[b][pl.ds(k * 16, 16)]
                    e16 = eidx_v[b][pl.ds(k * 16, 16)]
                    flat = (d16 - lo) * R2 + e16
                    inr = (d16 >= lo) & (d16 < hi)
                    dummy = _CDUM + k * 16 + lax.iota(jnp.int32, 16)
                    fidx_v[b][pl.ds(k * 16, 16)] = jnp.where(inr, flat, dummy)
                sdesc[j] = pltpu.async_copy(ones_v, accf.at[fidx_v[b]],
                                            ssem, add=True)
                if j + 2 < _CB_GRP:
                    idesc[j + 2] = issue_idx(j + 2)
            sdesc[_CB_GRP - 2].wait()
            sdesc[_CB_GRP - 1].wait()
            return 0
        lax.fori_loop(0, _CB_NGRP, group, 0)
        plsc.subcore_barrier()

        for cc in range(NC):
            @pl.when(c == cc)
            def _w(cc=cc, t=t):
                pltpu.sync_copy(accf.at[pl.ds(s * _CSUB, _CSUB)],
                                outs[t * NC + cc].at[pl.ds(s * _CSUB, _CSUB)])
        plsc.subcore_barrier()


@functools.lru_cache(maxsize=None)
def _cbuild_kernel():
    return pl.kernel(
        _cbuild_body,
        out_type=tuple(jax.ShapeDtypeStruct((_COUT,), jnp.float32)
                       for _ in range(TSTEPS * NC)),
        mesh=_sc_mesh(),
        scratch_types=(
            [pltpu.VMEM((_CB_CH,), jnp.int32)] * 6
            + [
                pltpu.VMEM((_CB_CH,), jnp.float32),
                pltpu.VMEM((_CZB,), jnp.float32),
                pltpu.VMEM_SHARED((_CACC,), jnp.float32),
                pltpu.SemaphoreType.DMA,
                pltpu.SemaphoreType.DMA,
                pltpu.SemaphoreType.DMA,
            ]
        ),
    )


def _cbuild(dst_flat, et_flat):
    outs = _cbuild_kernel()(dst_flat, et_flat)
    cmats = []
    for t in range(TSTEPS):
        halves = [outs[t * NC + cc][:_CFLAT].reshape(_CROWS, R2)
                  for cc in range(NC)]
        cmats.append(jnp.concatenate(halves, axis=0))
    return cmats


_CNT_BLK = 2000
_CNT_NBLK = EDGES // _CNT_BLK


def _cnt_body(et_ref, o_ref):
    i = pl.program_id(0)
    et = et_ref[0, 0, :]
    onehot = (et[:, None] ==
              lax.broadcasted_iota(jnp.int32, (1, R2), 1)).astype(jnp.float32)
    c = jnp.sum(onehot, axis=0, keepdims=True)
    val = jnp.broadcast_to(c[:, :, None], (1, R2, H))

    @pl.when(i % _CNT_NBLK == 0)
    def _init():
        o_ref[...] = jnp.zeros_like(o_ref)

    o_ref[...] = o_ref[...] + val


def _cnt_tc(et2d):
    return pl.pallas_call(
        _cnt_body,
        grid=(TSTEPS * _CNT_NBLK,),
        in_specs=[pl.BlockSpec((1, 1, _CNT_BLK), lambda i: (i, 0, 0))],
        out_specs=pl.BlockSpec((1, R2, H), lambda i: (i // _CNT_NBLK, 0, 0)),
        out_shape=jax.ShapeDtypeStruct((TSTEPS, R2, H), jnp.float32),
    )(et2d)

def _l2norm_body(x_ref, o_ref):
    x = x_ref[...]
    n = jnp.sqrt(jnp.sum(x * x, axis=1, keepdims=True))
    o_ref[...] = x / jnp.clip(n, 1e-12, None)


def _l2norm(x):
    bn = 1000
    return pl.pallas_call(
        _l2norm_body,
        grid=(NE // bn,),
        in_specs=[pl.BlockSpec((bn, H), lambda i: (i, 0))],
        out_specs=pl.BlockSpec((bn, H), lambda i: (i, 0)),
        out_shape=jax.ShapeDtypeStruct((NE, H), jnp.float32),
    )(x)


def _dotT(a, b):
    return lax.dot_general(a, b, (((1,), (1,)), ((), ())),
                           precision=lax.Precision.HIGHEST,
                           preferred_element_type=jnp.float32)


def _dot(a, b):
    return lax.dot_general(a, b, (((1,), (0,)), ((), ())),
                           precision=lax.Precision.HIGHEST,
                           preferred_element_type=jnp.float32)


def _gru_body(seg_ref, cnt_ref, emb_ref, h0_ref, wih_ref, whh_ref,
              bih_ref, bhh_ref, wn_ref, h0o_ref, h0w_ref):
    seg = seg_ref[0] + seg_ref[1]
    x_mean = seg / jnp.maximum(cnt_ref[...], 1.0)
    emb = emb_ref[...]
    h0 = h0_ref[...]
    x = jnp.concatenate([emb, x_mean], axis=1)
    gi = _dotT(x, wih_ref[...]) + bih_ref[...]
    gh = _dotT(h0, whh_ref[...]) + bhh_ref[...]
    i_r, i_z, i_n = gi[:, :H], gi[:, H:2 * H], gi[:, 2 * H:]
    h_r, h_z, h_n = gh[:, :H], gh[:, H:2 * H], gh[:, 2 * H:]
    r = jax.nn.sigmoid(i_r + h_r)
    z = jax.nn.sigmoid(i_z + h_z)
    n = jnp.tanh(i_n + r * h_n)
    h0n = (1.0 - z) * n + z * h0
    h0o_ref[...] = h0n
    h0w_ref[...] = _dot(h0n, wn_ref[...])


def _gru_step(seg_p, cnt_t, emb_rel, h0, w_ih, w_hh, b_ih, b_hh, w_n):
    return pl.pallas_call(
        _gru_body,
        out_shape=(
            jax.ShapeDtypeStruct((R2, H), jnp.float32),
            jax.ShapeDtypeStruct((R2, H), jnp.float32),
        ),
    )(seg_p, cnt_t, emb_rel, h0, w_ih, w_hh, b_ih, b_hh, w_n)


def _update_body(h_ref, s_ref, c_ref, h0w_ref, wn_ref, lw_ref, w1_ref, o_ref):
    h = h_ref[...]
    S = s_ref[0] + s_ref[1]
    R = _dot(c_ref[...], h0w_ref[...])
    agg = _dot(S, wn_ref[...]) + R
    loop = _dot(h, lw_ref[...])
    cur = agg + loop
    cur = jnp.where(cur >= 0, cur, cur * NEG_SLOPE)
    n = jnp.sqrt(jnp.sum(cur * cur, axis=1, keepdims=True))
    cur = cur / jnp.clip(n, 1e-12, None)
    tw = jax.nn.sigmoid(_dot(h, w1_ref[...]))
    o_ref[...] = tw * cur + (1.0 - tw) * h


def _update_step(h, s_p, cmat, h0w, w_n, loop_w, w1):
    bn = 1000
    return pl.pallas_call(
        _update_body,
        grid=(NE // bn,),
        in_specs=[
            pl.BlockSpec((bn, H), lambda i: (i, 0)),
            pl.BlockSpec((NC, bn, H), lambda i: (0, i, 0)),
            pl.BlockSpec((bn, R2), lambda i: (i, 0)),
            pl.BlockSpec((R2, H), lambda i: (0, 0)),
            pl.BlockSpec((H, H), lambda i: (0, 0)),
            pl.BlockSpec((H, H), lambda i: (0, 0)),
            pl.BlockSpec((H, H), lambda i: (0, 0)),
        ],
        out_specs=pl.BlockSpec((bn, H), lambda i: (i, 0)),
        out_shape=jax.ShapeDtypeStruct((NE, H), jnp.float32),
    )(h, s_p, cmat, h0w, w_n, loop_w, w1)


def kernel(edge_index, edge_type, use_cuda, dynamic_emb, emb_rel, w1,
           weight_neighbor, loop_weight, w_ih, w_hh, b_ih, b_hh):
    del use_cuda
    h = _l2norm(dynamic_emb)
    h0 = emb_rel
    b_ih2 = b_ih.reshape(1, 3 * H)
    b_hh2 = b_hh.reshape(1, 3 * H)
    cnts = _cnt_tc(edge_type.reshape(TSTEPS * _CNT_NBLK, 1, _CNT_BLK))
    cmats = _cbuild(edge_index[:, 1, :].reshape(-1), edge_type.reshape(-1))
    outs = []
    for t in range(TSTEPS):
        src = edge_index[t, 0]
        dst = edge_index[t, 1]
        et = edge_type[t]
        seg_p, s_p = _fused_sweep(h, src, et, dst)
        h0, h0w = _gru_step(seg_p, cnts[t], emb_rel, h0, w_ih, w_hh,
                            b_ih2, b_hh2, weight_neighbor)
        h = _update_step(h, s_p, cmats[t], h0w, weight_neighbor,
                         loop_weight, w1)
        outs.append(h)
    return jnp.stack(outs, axis=0)


# speedup vs baseline: 9.0231x; 1.0008x over previous
import functools

import jax
import jax.numpy as jnp
from jax import lax
from jax.experimental import pallas as pl
from jax.experimental.pallas import tpu as pltpu
from jax.experimental.pallas import tpu_sc as plsc

NE = 10000
H = 128
EDGES = 320000
R2 = 400
TSTEPS = 3

NC = 2
NS = 16
NW = NC * NS
EW = EDGES // NW
CH = 80
NCHUNK = EW // CH
ZB = 48
NB = 5
NR = 3

_REL_PER_SUB = 40
_ENT_PER_SUB = 624

NEG_SLOPE = (1.0 / 8.0 + 1.0 / 3.0) / 2.0


def _sc_mesh():
    return plsc.VectorSubcoreMesh(
        core_axis_name="c", subcore_axis_name="s", num_cores=NC, num_subcores=NS
    )


def _zero_vmem(ref, nrows, ncols):
    def row(i, _):
        def col(j, __):
            ref[i, pl.ds(j * 16, 16)] = jnp.zeros((16,), jnp.float32)
            return 0
        lax.fori_loop(0, ncols // 16, col, 0)
        return 0
    lax.fori_loop(0, nrows, row, 0)


def _ent_copy(s, fn):
    for r in range(_ENT_PER_SUB // ZB):
        fn(s * _ENT_PER_SUB + r * ZB, ZB)

    @pl.when(s == NS - 1)
    def _tail():
        fn(NS * _ENT_PER_SUB, NE - NS * _ENT_PER_SUB)


def _fused_sweep_body(table, src, et, dst, seg_out, s_out, *scr):
    gidx_v = scr[0:NB]
    eidx_v = scr[NB:2 * NB]
    didx_v = scr[2 * NB:3 * NB]
    rows_v = scr[3 * NB:3 * NB + NR]
    zbuf_v, seg_sh, s_sh, isem, gsem, ssem = scr[3 * NB + NR:]
    c = lax.axis_index("c")
    s = lax.axis_index("s")
    wid = c * NS + s
    base = wid * EW

    _zero_vmem(zbuf_v, ZB, H)

    @pl.when(s < R2 // _REL_PER_SUB)
    def _zrel():
        pltpu.sync_copy(zbuf_v.at[pl.ds(0, _REL_PER_SUB)],
                        seg_sh.at[pl.ds(s * _REL_PER_SUB, _REL_PER_SUB)])

    def _zent(o, sz):
        pltpu.sync_copy(zbuf_v.at[pl.ds(0, sz)], s_sh.at[pl.ds(o, sz)])
    _ent_copy(s, _zent)
    plsc.subcore_barrier()

    def group(g, _):
        gb = base + g * (NB * CH)
        idesc = []
        for b in range(NB):
            off = gb + b * CH
            idesc.append([
                pltpu.async_copy(src.at[pl.ds(off, CH)], gidx_v[b], isem),
                pltpu.async_copy(et.at[pl.ds(off, CH)], eidx_v[b], isem),
                pltpu.async_copy(dst.at[pl.ds(off, CH)], didx_v[b], isem),
            ])
        gdesc = [None] * NB
        sdesc = [None] * NB
        for b in range(NB):
            rb = b % NR
            if b >= NR:
                for d in sdesc[b - NR]:
                    d.wait()
            for d in idesc[b]:
                d.wait()
            gdesc[b] = pltpu.async_copy(table.at[gidx_v[b]], rows_v[rb], gsem)
            if b >= 1:
                gdesc[b - 1].wait()
                pb = (b - 1) % NR
                sdesc[b - 1] = [
                    pltpu.async_copy(rows_v[pb], seg_sh.at[eidx_v[b - 1]],
                                     ssem, add=True),
                    pltpu.async_copy(rows_v[pb], s_sh.at[didx_v[b - 1]],
                                     ssem, add=True),
                ]
        gdesc[NB - 1].wait()
        lb = (NB - 1) % NR
        sdesc[NB - 1] = [
            pltpu.async_copy(rows_v[lb], seg_sh.at[eidx_v[NB - 1]],
                             ssem, add=True),
            pltpu.async_copy(rows_v[lb], s_sh.at[didx_v[NB - 1]],
                             ssem, add=True),
        ]
        for b in range(max(NB - NR, 0), NB):
            for d in sdesc[b]:
                d.wait()
        return 0
    lax.fori_loop(0, NCHUNK // NB, group, 0)
    plsc.subcore_barrier()

    @pl.when(s < R2 // _REL_PER_SUB)
    def _orel():
        pltpu.sync_copy(seg_sh.at[pl.ds(s * _REL_PER_SUB, _REL_PER_SUB)],
                        seg_out.at[c, pl.ds(s * _REL_PER_SUB, _REL_PER_SUB)])

    def _oent(o, sz):
        pltpu.sync_copy(s_sh.at[pl.ds(o, sz)], s_out.at[c, pl.ds(o, sz)])
    _ent_copy(s, _oent)


@functools.lru_cache(maxsize=None)
def _fused_sweep_kernel():
    return pl.kernel(
        _fused_sweep_body,
        out_type=(
            jax.ShapeDtypeStruct((NC, R2, H), jnp.float32),
            jax.ShapeDtypeStruct((NC, NE, H), jnp.float32),
        ),
        mesh=_sc_mesh(),
        scratch_types=(
            [pltpu.VMEM((CH,), jnp.int32)] * (3 * NB)
            + [pltpu.VMEM((CH, H), jnp.float32)] * NR
            + [
                pltpu.VMEM((ZB, H), jnp.float32),
                pltpu.VMEM_SHARED((R2, H), jnp.float32),
                pltpu.VMEM_SHARED((NE, H), jnp.float32),
                pltpu.SemaphoreType.DMA,
                pltpu.SemaphoreType.DMA,
                pltpu.SemaphoreType.DMA,
            ]
        ),
    )


def _fused_sweep(table, src, et, dst):
    return _fused_sweep_kernel()(table, src, et, dst)


_CB_LD = 2000
_CB_CH = 80
_CB_SPL = _CB_LD // _CB_CH
_CB_EPT = EDGES // NS
_CB_NLD = _CB_EPT // _CB_LD
_CB_NF = 4
_CROWS = NE // NC
_CFLAT = _CROWS * R2
_CSUB = 125056
_COUT = _CSUB * NS
_CDUM = _COUT
_CZB = 512
_CNZ = 245
_CACC = _CZB * _CNZ * NS


def _cbuild_body(dstf, etf, *args):
    outs = args[:TSTEPS * NC]
    didx_v = args[TSTEPS * NC]
    eidx_v = args[TSTEPS * NC + 1]
    fidx_v = args[TSTEPS * NC + 2:TSTEPS * NC + 2 + _CB_NF]
    (ones_v, zbuf_v, accf, isem, ssem, zsem) = args[TSTEPS * NC + 2 + _CB_NF:]
    c = lax.axis_index("c")
    s = lax.axis_index("s")
    lo = c * _CROWS
    hi = lo + _CROWS

    def f1(k, _):
        ones_v[pl.ds(k * 16, 16)] = jnp.ones((16,), jnp.float32)
        return 0
    lax.fori_loop(0, _CB_CH // 16, f1, 0)

    def fz(k, _):
        zbuf_v[pl.ds(k * 16, 16)] = jnp.zeros((16,), jnp.float32)
        return 0
    lax.fori_loop(0, _CZB // 16, fz, 0)

    for t in range(TSTEPS):
        zd = []
        for r in range(_CNZ):
            o = (s * _CNZ + r) * _CZB
            zd.append(pltpu.async_copy(zbuf_v, accf.at[pl.ds(o, _CZB)], zsem))
        for d in zd:
            d.wait()
        plsc.subcore_barrier()

        toff = t * EDGES + s * _CB_EPT

        def bigstep(g, _, toff=toff):
            off = toff + g * _CB_LD
            l1 = pltpu.async_copy(dstf.at[pl.ds(off, _CB_LD)], didx_v, isem)
            l2 = pltpu.async_copy(etf.at[pl.ds(off, _CB_LD)], eidx_v, isem)
            l1.wait()
            l2.wait()
            sd = [None] * _CB_SPL
            for j in range(_CB_SPL):
                f = j % _CB_NF
                if j >= _CB_NF:
                    sd[j - _CB_NF].wait()
                for k in range(_CB_CH // 16):
                    p = j * _CB_CH + k * 16
                    d16 = didx_v[pl.ds(p, 16)]
                    e16 = eidx_v[pl.ds(p, 16)]
                    flat = (d16 - lo) * R2 + e16
                    inr = (d16 >= lo) & (d16 < hi)
                    dummy = _CDUM + k * 16 + lax.iota(jnp.int32, 16)
                    fidx_v[f][pl.ds(k * 16, 16)] = jnp.where(inr, flat, dummy)
                sd[j] = pltpu.async_copy(ones_v, accf.at[fidx_v[f]],
                                         ssem, add=True)
            for j in range(_CB_SPL - _CB_NF, _CB_SPL):
                sd[j].wait()
            return 0
        lax.fori_loop(0, _CB_NLD, bigstep, 0)
        plsc.subcore_barrier()

        for cc in range(NC):
            @pl.when(c == cc)
            def _w(cc=cc, t=t):
                pltpu.sync_copy(accf.at[pl.ds(s * _CSUB, _CSUB)],
                                outs[t * NC + cc].at[pl.ds(s * _CSUB,
                                                           _CSUB)])
        plsc.subcore_barrier()


@functools.lru_cache(maxsize=None)
def _cbuild_kernel():
    return pl.kernel(
        _cbuild_body,
        out_type=tuple(jax.ShapeDtypeStruct((_COUT,), jnp.float32)
                       for _ in range(TSTEPS * NC)),
        mesh=_sc_mesh(),
        scratch_types=(
            [pltpu.VMEM((_CB_LD,), jnp.int32)] * 2
            + [pltpu.VMEM((_CB_CH,), jnp.int32)] * _CB_NF
            + [
                pltpu.VMEM((_CB_CH,), jnp.float32),
                pltpu.VMEM((_CZB,), jnp.float32),
                pltpu.VMEM_SHARED((_CACC,), jnp.float32),
                pltpu.SemaphoreType.DMA,
                pltpu.SemaphoreType.DMA,
                pltpu.SemaphoreType.DMA,
            ]
        ),
    )


def _cbuild(dst_flat, et_flat):
    outs = _cbuild_kernel()(dst_flat, et_flat)
    cmats = []
    for t in range(TSTEPS):
        halves = [outs[t * NC + cc][:_CFLAT].reshape(_CROWS, R2)
                  for cc in range(NC)]
        cmats.append(jnp.concatenate(halves, axis=0))
    return cmats


_CNT_BLK = 2000
_CNT_NBLK = EDGES // _CNT_BLK


def _cnt_body(et_ref, o_ref):
    i = pl.program_id(0)
    et = et_ref[0, 0, :]
    onehot = (et[:, None] ==
              lax.broadcasted_iota(jnp.int32, (1, R2), 1)).astype(jnp.float32)
    c = jnp.sum(onehot, axis=0, keepdims=True)
    val = jnp.broadcast_to(c[:, :, None], (1, R2, H))

    @pl.when(i % _CNT_NBLK == 0)
    def _init():
        o_ref[...] = jnp.zeros_like(o_ref)

    o_ref[...] = o_ref[...] + val


def _cnt_tc(et2d):
    return pl.pallas_call(
        _cnt_body,
        grid=(TSTEPS * _CNT_NBLK,),
        in_specs=[pl.BlockSpec((1, 1, _CNT_BLK), lambda i: (i, 0, 0))],
        out_specs=pl.BlockSpec((1, R2, H), lambda i: (i // _CNT_NBLK, 0, 0)),
        out_shape=jax.ShapeDtypeStruct((TSTEPS, R2, H), jnp.float32),
    )(et2d)

def _l2norm_body(x_ref, o_ref):
    x = x_ref[...]
    n = jnp.sqrt(jnp.sum(x * x, axis=1, keepdims=True))
    o_ref[...] = x / jnp.clip(n, 1e-12, None)


def _l2norm(x):
    bn = 1000
    return pl.pallas_call(
        _l2norm_body,
        grid=(NE // bn,),
        in_specs=[pl.BlockSpec((bn, H), lambda i: (i, 0))],
        out_specs=pl.BlockSpec((bn, H), lambda i: (i, 0)),
        out_shape=jax.ShapeDtypeStruct((NE, H), jnp.float32),
    )(x)


def _dotT(a, b):
    return lax.dot_general(a, b, (((1,), (1,)), ((), ())),
                           precision=lax.Precision.HIGHEST,
                           preferred_element_type=jnp.float32)


def _dot(a, b):
    return lax.dot_general(a, b, (((1,), (0,)), ((), ())),
                           precision=lax.Precision.HIGHEST,
                           preferred_element_type=jnp.float32)


def _gru_body(seg_ref, cnt_ref, emb_ref, h0_ref, wih_ref, whh_ref,
              bih_ref, bhh_ref, wn_ref, h0o_ref, h0w_ref):
    seg = seg_ref[0] + seg_ref[1]
    x_mean = seg / jnp.maximum(cnt_ref[...], 1.0)
    emb = emb_ref[...]
    h0 = h0_ref[...]
    x = jnp.concatenate([emb, x_mean], axis=1)
    gi = _dotT(x, wih_ref[...]) + bih_ref[...]
    gh = _dotT(h0, whh_ref[...]) + bhh_ref[...]
    i_r, i_z, i_n = gi[:, :H], gi[:, H:2 * H], gi[:, 2 * H:]
    h_r, h_z, h_n = gh[:, :H], gh[:, H:2 * H], gh[:, 2 * H:]
    r = jax.nn.sigmoid(i_r + h_r)
    z = jax.nn.sigmoid(i_z + h_z)
    n = jnp.tanh(i_n + r * h_n)
    h0n = (1.0 - z) * n + z * h0
    h0o_ref[...] = h0n
    h0w_ref[...] = _dot(h0n, wn_ref[...])


def _gru_step(seg_p, cnt_t, emb_rel, h0, w_ih, w_hh, b_ih, b_hh, w_n):
    return pl.pallas_call(
        _gru_body,
        out_shape=(
            jax.ShapeDtypeStruct((R2, H), jnp.float32),
            jax.ShapeDtypeStruct((R2, H), jnp.float32),
        ),
    )(seg_p, cnt_t, emb_rel, h0, w_ih, w_hh, b_ih, b_hh, w_n)


def _update_body(h_ref, s_ref, c_ref, h0w_ref, wn_ref, lw_ref, w1_ref, o_ref):
    h = h_ref[...]
    S = s_ref[0] + s_ref[1]
    R = _dot(c_ref[...], h0w_ref[...])
    agg = _dot(S, wn_ref[...]) + R
    loop = _dot(h, lw_ref[...])
    cur = agg + loop
    cur = jnp.where(cur >= 0, cur, cur * NEG_SLOPE)
    n = jnp.sqrt(jnp.sum(cur * cur, axis=1, keepdims=True))
    cur = cur / jnp.clip(n, 1e-12, None)
    tw = jax.nn.sigmoid(_dot(h, w1_ref[...]))
    o_ref[...] = tw * cur + (1.0 - tw) * h


def _update_step(h, s_p, cmat, h0w, w_n, loop_w, w1):
    bn = 1000
    return pl.pallas_call(
        _update_body,
        grid=(NE // bn,),
        in_specs=[
            pl.BlockSpec((bn, H), lambda i: (i, 0)),
            pl.BlockSpec((NC, bn, H), lambda i: (0, i, 0)),
            pl.BlockSpec((bn, R2), lambda i: (i, 0)),
            pl.BlockSpec((R2, H), lambda i: (0, 0)),
            pl.BlockSpec((H, H), lambda i: (0, 0)),
            pl.BlockSpec((H, H), lambda i: (0, 0)),
            pl.BlockSpec((H, H), lambda i: (0, 0)),
        ],
        out_specs=pl.BlockSpec((bn, H), lambda i: (i, 0)),
        out_shape=jax.ShapeDtypeStruct((NE, H), jnp.float32),
    )(h, s_p, cmat, h0w, w_n, loop_w, w1)


def kernel(edge_index, edge_type, use_cuda, dynamic_emb, emb_rel, w1,
           weight_neighbor, loop_weight, w_ih, w_hh, b_ih, b_hh):
    del use_cuda
    h = _l2norm(dynamic_emb)
    h0 = emb_rel
    b_ih2 = b_ih.reshape(1, 3 * H)
    b_hh2 = b_hh.reshape(1, 3 * H)
    cnts = _cnt_tc(edge_type.reshape(TSTEPS * _CNT_NBLK, 1, _CNT_BLK))
    cmats = _cbuild(edge_index[:, 1, :].reshape(-1), edge_type.reshape(-1))
    outs = []
    for t in range(TSTEPS):
        src = edge_index[t, 0]
        dst = edge_index[t, 1]
        et = edge_type[t]
        seg_p, s_p = _fused_sweep(h, src, et, dst)
        h0, h0w = _gru_step(seg_p, cnts[t], emb_rel, h0, w_ih, w_hh,
                            b_ih2, b_hh2, weight_neighbor)
        h = _update_step(h, s_p, cmats[t], h0w, weight_neighbor,
                         loop_weight, w1)
        outs.append(h)
    return jnp.stack(outs, axis=0)


# speedup vs baseline: 9.3079x; 1.0316x over previous
import functools

import jax
import jax.numpy as jnp
from jax import lax
from jax.experimental import pallas as pl
from jax.experimental.pallas import tpu as pltpu
from jax.experimental.pallas import tpu_sc as plsc

NE = 10000
H = 128
EDGES = 320000
R2 = 400
TSTEPS = 3

NC = 2
NS = 16
NW = NC * NS
EW = EDGES // NW
CH = 80
NCHUNK = EW // CH
ZB = 48
NB = 25
NR = 3

_REL_PER_SUB = 40
_ENT_PER_SUB = 624

NEG_SLOPE = (1.0 / 8.0 + 1.0 / 3.0) / 2.0


def _sc_mesh():
    return plsc.VectorSubcoreMesh(
        core_axis_name="c", subcore_axis_name="s", num_cores=NC, num_subcores=NS
    )


def _zero_vmem(ref, nrows, ncols):
    def row(i, _):
        def col(j, __):
            ref[i, pl.ds(j * 16, 16)] = jnp.zeros((16,), jnp.float32)
            return 0
        lax.fori_loop(0, ncols // 16, col, 0)
        return 0
    lax.fori_loop(0, nrows, row, 0)


def _ent_copy(s, fn):
    for r in range(_ENT_PER_SUB // ZB):
        fn(s * _ENT_PER_SUB + r * ZB, ZB)

    @pl.when(s == NS - 1)
    def _tail():
        fn(NS * _ENT_PER_SUB, NE - NS * _ENT_PER_SUB)


def _fused_sweep_body(table, src, et, dst, seg_out, s_out, *scr):
    gidx_v = scr[0:NB]
    eidx_v = scr[NB:2 * NB]
    didx_v = scr[2 * NB:3 * NB]
    rows_v = scr[3 * NB:3 * NB + NR]
    zbuf_v, seg_sh, s_sh, isem, gsem, ssem = scr[3 * NB + NR:]
    c = lax.axis_index("c")
    s = lax.axis_index("s")
    wid = c * NS + s
    base = wid * EW

    _zero_vmem(zbuf_v, ZB, H)

    @pl.when(s < R2 // _REL_PER_SUB)
    def _zrel():
        pltpu.sync_copy(zbuf_v.at[pl.ds(0, _REL_PER_SUB)],
                        seg_sh.at[pl.ds(s * _REL_PER_SUB, _REL_PER_SUB)])

    def _zent(o, sz):
        pltpu.sync_copy(zbuf_v.at[pl.ds(0, sz)], s_sh.at[pl.ds(o, sz)])
    _ent_copy(s, _zent)
    plsc.subcore_barrier()

    def group(g, _):
        gb = base + g * (NB * CH)
        idesc = []
        for b in range(NB):
            off = gb + b * CH
            idesc.append([
                pltpu.async_copy(src.at[pl.ds(off, CH)], gidx_v[b], isem),
                pltpu.async_copy(et.at[pl.ds(off, CH)], eidx_v[b], isem),
                pltpu.async_copy(dst.at[pl.ds(off, CH)], didx_v[b], isem),
            ])
        gdesc = [None] * NB
        sdesc = [None] * NB
        for b in range(NB):
            rb = b % NR
            if b >= NR:
                for d in sdesc[b - NR]:
                    d.wait()
            for d in idesc[b]:
                d.wait()
            gdesc[b] = pltpu.async_copy(table.at[gidx_v[b]], rows_v[rb], gsem)
            if b >= 1:
                gdesc[b - 1].wait()
                pb = (b - 1) % NR
                sdesc[b - 1] = [
                    pltpu.async_copy(rows_v[pb], seg_sh.at[eidx_v[b - 1]],
                                     ssem, add=True),
                    pltpu.async_copy(rows_v[pb], s_sh.at[didx_v[b - 1]],
                                     ssem, add=True),
                ]
        gdesc[NB - 1].wait()
        lb = (NB - 1) % NR
        sdesc[NB - 1] = [
            pltpu.async_copy(rows_v[lb], seg_sh.at[eidx_v[NB - 1]],
                             ssem, add=True),
            pltpu.async_copy(rows_v[lb], s_sh.at[didx_v[NB - 1]],
                             ssem, add=True),
        ]
        for b in range(max(NB - NR, 0), NB):
            for d in sdesc[b]:
                d.wait()
        return 0
    lax.fori_loop(0, NCHUNK // NB, group, 0)
    plsc.subcore_barrier()

    @pl.when(s < R2 // _REL_PER_SUB)
    def _orel():
        pltpu.sync_copy(seg_sh.at[pl.ds(s * _REL_PER_SUB, _REL_PER_SUB)],
                        seg_out.at[c, pl.ds(s * _REL_PER_SUB, _REL_PER_SUB)])

    def _oent(o, sz):
        pltpu.sync_copy(s_sh.at[pl.ds(o, sz)], s_out.at[c, pl.ds(o, sz)])
    _ent_copy(s, _oent)


@functools.lru_cache(maxsize=None)
def _fused_sweep_kernel():
    return pl.kernel(
        _fused_sweep_body,
        out_type=(
            jax.ShapeDtypeStruct((NC, R2, H), jnp.float32),
            jax.ShapeDtypeStruct((NC, NE, H), jnp.float32),
        ),
        mesh=_sc_mesh(),
        scratch_types=(
            [pltpu.VMEM((CH,), jnp.int32)] * (3 * NB)
            + [pltpu.VMEM((CH, H), jnp.float32)] * NR
            + [
                pltpu.VMEM((ZB, H), jnp.float32),
                pltpu.VMEM_SHARED((R2, H), jnp.float32),
                pltpu.VMEM_SHARED((NE, H), jnp.float32),
                pltpu.SemaphoreType.DMA,
                pltpu.SemaphoreType.DMA,
                pltpu.SemaphoreType.DMA,
            ]
        ),
    )


def _fused_sweep(table, src, et, dst):
    return _fused_sweep_kernel()(table, src, et, dst)


_CB_LD = 2000
_CB_CH = 80
_CB_SPL = _CB_LD // _CB_CH
_CB_EPT = EDGES // NS
_CB_NLD = _CB_EPT // _CB_LD
_CB_NF = 4
_CROWS = NE // NC
_CFLAT = _CROWS * R2
_CSUB = 125056
_COUT = _CSUB * NS
_CDUM = _COUT
_CZB = 512
_CNZ = 245
_CACC = _CZB * _CNZ * NS


def _cbuild_body(dstf, etf, *args):
    outs = args[:TSTEPS * NC]
    didx_v = args[TSTEPS * NC]
    eidx_v = args[TSTEPS * NC + 1]
    fidx_v = args[TSTEPS * NC + 2:TSTEPS * NC + 2 + _CB_NF]
    (ones_v, zbuf_v, accf, isem, ssem, zsem) = args[TSTEPS * NC + 2 + _CB_NF:]
    c = lax.axis_index("c")
    s = lax.axis_index("s")
    lo = c * _CROWS
    hi = lo + _CROWS

    def f1(k, _):
        ones_v[pl.ds(k * 16, 16)] = jnp.ones((16,), jnp.float32)
        return 0
    lax.fori_loop(0, _CB_CH // 16, f1, 0)

    def fz(k, _):
        zbuf_v[pl.ds(k * 16, 16)] = jnp.zeros((16,), jnp.float32)
        return 0
    lax.fori_loop(0, _CZB // 16, fz, 0)

    for t in range(TSTEPS):
        zd = []
        for r in range(_CNZ):
            o = (s * _CNZ + r) * _CZB
            zd.append(pltpu.async_copy(zbuf_v, accf.at[pl.ds(o, _CZB)], zsem))
        for d in zd:
            d.wait()
        plsc.subcore_barrier()

        toff = t * EDGES + s * _CB_EPT

        def bigstep(g, _, toff=toff):
            off = toff + g * _CB_LD
            l1 = pltpu.async_copy(dstf.at[pl.ds(off, _CB_LD)], didx_v, isem)
            l2 = pltpu.async_copy(etf.at[pl.ds(off, _CB_LD)], eidx_v, isem)
            l1.wait()
            l2.wait()
            sd = [None] * _CB_SPL
            for j in range(_CB_SPL):
                f = j % _CB_NF
                if j >= _CB_NF:
                    sd[j - _CB_NF].wait()
                for k in range(_CB_CH // 16):
                    p = j * _CB_CH + k * 16
                    d16 = didx_v[pl.ds(p, 16)]
                    e16 = eidx_v[pl.ds(p, 16)]
                    flat = (d16 - lo) * R2 + e16
                    inr = (d16 >= lo) & (d16 < hi)
                    dummy = _CDUM + k * 16 + lax.iota(jnp.int32, 16)
                    fidx_v[f][pl.ds(k * 16, 16)] = jnp.where(inr, flat, dummy)
                sd[j] = pltpu.async_copy(ones_v, accf.at[fidx_v[f]],
                                         ssem, add=True)
            for j in range(_CB_SPL - _CB_NF, _CB_SPL):
                sd[j].wait()
            return 0
        lax.fori_loop(0, _CB_NLD, bigstep, 0)
        plsc.subcore_barrier()

        for cc in range(NC):
            @pl.when(c == cc)
            def _w(cc=cc, t=t):
                pltpu.sync_copy(accf.at[pl.ds(s * _CSUB, _CSUB)],
                                outs[t * NC + cc].at[pl.ds(s * _CSUB,
                                                           _CSUB)])
        plsc.subcore_barrier()


@functools.lru_cache(maxsize=None)
def _cbuild_kernel():
    return pl.kernel(
        _cbuild_body,
        out_type=tuple(jax.ShapeDtypeStruct((_COUT,), jnp.float32)
                       for _ in range(TSTEPS * NC)),
        mesh=_sc_mesh(),
        scratch_types=(
            [pltpu.VMEM((_CB_LD,), jnp.int32)] * 2
            + [pltpu.VMEM((_CB_CH,), jnp.int32)] * _CB_NF
            + [
                pltpu.VMEM((_CB_CH,), jnp.float32),
                pltpu.VMEM((_CZB,), jnp.float32),
                pltpu.VMEM_SHARED((_CACC,), jnp.float32),
                pltpu.SemaphoreType.DMA,
                pltpu.SemaphoreType.DMA,
                pltpu.SemaphoreType.DMA,
            ]
        ),
    )


def _cbuild(dst_flat, et_flat):
    outs = _cbuild_kernel()(dst_flat, et_flat)
    cmats = []
    for t in range(TSTEPS):
        halves = [outs[t * NC + cc][:_CFLAT].reshape(_CROWS, R2)
                  for cc in range(NC)]
        cmats.append(jnp.concatenate(halves, axis=0))
    return cmats


_CNT_BLK = 2000
_CNT_NBLK = EDGES // _CNT_BLK


def _cnt_body(et_ref, o_ref):
    i = pl.program_id(0)
    et = et_ref[0, 0, :]
    onehot = (et[:, None] ==
              lax.broadcasted_iota(jnp.int32, (1, R2), 1)).astype(jnp.float32)
    c = jnp.sum(onehot, axis=0, keepdims=True)
    val = jnp.broadcast_to(c[:, :, None], (1, R2, H))

    @pl.when(i % _CNT_NBLK == 0)
    def _init():
        o_ref[...] = jnp.zeros_like(o_ref)

    o_ref[...] = o_ref[...] + val


def _cnt_tc(et2d):
    return pl.pallas_call(
        _cnt_body,
        grid=(TSTEPS * _CNT_NBLK,),
        in_specs=[pl.BlockSpec((1, 1, _CNT_BLK), lambda i: (i, 0, 0))],
        out_specs=pl.BlockSpec((1, R2, H), lambda i: (i // _CNT_NBLK, 0, 0)),
        out_shape=jax.ShapeDtypeStruct((TSTEPS, R2, H), jnp.float32),
    )(et2d)

def _l2norm_body(x_ref, o_ref):
    x = x_ref[...]
    n = jnp.sqrt(jnp.sum(x * x, axis=1, keepdims=True))
    o_ref[...] = x / jnp.clip(n, 1e-12, None)


def _l2norm(x):
    bn = 1000
    return pl.pallas_call(
        _l2norm_body,
        grid=(NE // bn,),
        in_specs=[pl.BlockSpec((bn, H), lambda i: (i, 0))],
        out_specs=pl.BlockSpec((bn, H), lambda i: (i, 0)),
        out_shape=jax.ShapeDtypeStruct((NE, H), jnp.float32),
    )(x)


def _dotT(a, b):
    return lax.dot_general(a, b, (((1,), (1,)), ((), ())),
                           precision=lax.Precision.HIGHEST,
                           preferred_element_type=jnp.float32)


def _dot(a, b):
    return lax.dot_general(a, b, (((1,), (0,)), ((), ())),
                           precision=lax.Precision.HIGHEST,
                           preferred_element_type=jnp.float32)


def _gru_body(seg_ref, cnt_ref, emb_ref, h0_ref, wih_ref, whh_ref,
              bih_ref, bhh_ref, wn_ref, h0o_ref, h0w_ref):
    seg = seg_ref[0] + seg_ref[1]
    x_mean = seg / jnp.maximum(cnt_ref[...], 1.0)
    emb = emb_ref[...]
    h0 = h0_ref[...]
    x = jnp.concatenate([emb, x_mean], axis=1)
    gi = _dotT(x, wih_ref[...]) + bih_ref[...]
    gh = _dotT(h0, whh_ref[...]) + bhh_ref[...]
    i_r, i_z, i_n = gi[:, :H], gi[:, H:2 * H], gi[:, 2 * H:]
    h_r, h_z, h_n = gh[:, :H], gh[:, H:2 * H], gh[:, 2 * H:]
    r = jax.nn.sigmoid(i_r + h_r)
    z = jax.nn.sigmoid(i_z + h_z)
    n = jnp.tanh(i_n + r * h_n)
    h0n = (1.0 - z) * n + z * h0
    h0o_ref[...] = h0n
    h0w_ref[...] = _dot(h0n, wn_ref[...])


def _gru_step(seg_p, cnt_t, emb_rel, h0, w_ih, w_hh, b_ih, b_hh, w_n):
    return pl.pallas_call(
        _gru_body,
        out_shape=(
            jax.ShapeDtypeStruct((R2, H), jnp.float32),
            jax.ShapeDtypeStruct((R2, H), jnp.float32),
        ),
    )(seg_p, cnt_t, emb_rel, h0, w_ih, w_hh, b_ih, b_hh, w_n)


def _update_body(h_ref, s_ref, c_ref, h0w_ref, wn_ref, lw_ref, w1_ref, o_ref):
    h = h_ref[...]
    S = s_ref[0] + s_ref[1]
    R = _dot(c_ref[...], h0w_ref[...])
    agg = _dot(S, wn_ref[...]) + R
    loop = _dot(h, lw_ref[...])
    cur = agg + loop
    cur = jnp.where(cur >= 0, cur, cur * NEG_SLOPE)
    n = jnp.sqrt(jnp.sum(cur * cur, axis=1, keepdims=True))
    cur = cur / jnp.clip(n, 1e-12, None)
    tw = jax.nn.sigmoid(_dot(h, w1_ref[...]))
    o_ref[...] = tw * cur + (1.0 - tw) * h


def _update_step(h, s_p, cmat, h0w, w_n, loop_w, w1):
    bn = 1000
    return pl.pallas_call(
        _update_body,
        grid=(NE // bn,),
        in_specs=[
            pl.BlockSpec((bn, H), lambda i: (i, 0)),
            pl.BlockSpec((NC, bn, H), lambda i: (0, i, 0)),
            pl.BlockSpec((bn, R2), lambda i: (i, 0)),
            pl.BlockSpec((R2, H), lambda i: (0, 0)),
            pl.BlockSpec((H, H), lambda i: (0, 0)),
            pl.BlockSpec((H, H), lambda i: (0, 0)),
            pl.BlockSpec((H, H), lambda i: (0, 0)),
        ],
        out_specs=pl.BlockSpec((bn, H), lambda i: (i, 0)),
        out_shape=jax.ShapeDtypeStruct((NE, H), jnp.float32),
    )(h, s_p, cmat, h0w, w_n, loop_w, w1)


def kernel(edge_index, edge_type, use_cuda, dynamic_emb, emb_rel, w1,
           weight_neighbor, loop_weight, w_ih, w_hh, b_ih, b_hh):
    del use_cuda
    h = _l2norm(dynamic_emb)
    h0 = emb_rel
    b_ih2 = b_ih.reshape(1, 3 * H)
    b_hh2 = b_hh.reshape(1, 3 * H)
    cnts = _cnt_tc(edge_type.reshape(TSTEPS * _CNT_NBLK, 1, _CNT_BLK))
    cmats = _cbuild(edge_index[:, 1, :].reshape(-1), edge_type.reshape(-1))
    outs = []
    for t in range(TSTEPS):
        src = edge_index[t, 0]
        dst = edge_index[t, 1]
        et = edge_type[t]
        seg_p, s_p = _fused_sweep(h, src, et, dst)
        h0, h0w = _gru_step(seg_p, cnts[t], emb_rel, h0, w_ih, w_hh,
                            b_ih2, b_hh2, weight_neighbor)
        h = _update_step(h, s_p, cmats[t], h0w, weight_neighbor,
                         loop_weight, w1)
        outs.append(h)
    return jnp.stack(outs, axis=0)
